# baseline (speedup 1.0000x reference)
import functools

import jax
import jax.numpy as jnp
from jax import lax
from jax.experimental import pallas as pl
from jax.experimental.pallas import tpu as pltpu
from jax.experimental.pallas import tpu_sc as plsc

B, L, V, D = 16384, 50, 1000000, 64
NW = 32
PAIRS = B // 2
PAIRS_PER_W = PAIRS // NW
TOK_PER_PAIR = 2 * L
TOK_PER_W = PAIRS_PER_W * TOK_PER_PAIR
CHUNK = 128
N_CHUNKS = TOK_PER_W // CHUNK

_BV = 65536


def _tc_body(w_ref, e_ref, o_ref):
    o_ref[...] = lax.dot_general(
        w_ref[...], e_ref[...], (((0,), (0,)), ((), ())),
        preferred_element_type=jnp.float32)


def _tc_project(emb, wr):
    embT = emb.T
    grid = (V + _BV - 1) // _BV
    return pl.pallas_call(
        _tc_body,
        grid=(grid,),
        in_specs=[
            pl.BlockSpec((D, 2), lambda i: (0, 0)),
            pl.BlockSpec((D, _BV), lambda i: (0, i)),
        ],
        out_specs=pl.BlockSpec((2, _BV), lambda i: (0, i)),
        out_shape=jax.ShapeDtypeStruct((2, V), jnp.float32),
    )(wr, embT)


def _sc_body(x_hbm, t2_hbm, b_hbm, out_hbm,
             idx_v, buf_v, out_v, b_v, sem):
    wid = lax.axis_index("s") * 2 + lax.axis_index("c")
    pltpu.sync_copy(x_hbm.at[wid], idx_v)
    pltpu.sync_copy(b_hbm, b_v)

    def _fire8(g, _):
        cps = []
        for k in range(8):
            c = g * 8 + k
            cps.append(pltpu.async_copy(
                t2_hbm.at[idx_v.at[pl.ds(c * CHUNK, CHUNK)]],
                buf_v.at[pl.ds(c * CHUNK, CHUNK)], sem))
        for cp in cps:
            cp.wait()
        return 0
    lax.fori_loop(0, N_CHUNKS // 8, _fire8, 0)

    def _tok(j, accs):
        base = j * PAIRS_PER_W
        return tuple(a + buf_v[pl.ds(base + cc * 16, 16)]
                     for cc, a in enumerate(accs))

    accs = lax.fori_loop(
        0, TOK_PER_PAIR, _tok,
        tuple(jnp.zeros((16,), jnp.float32) for _ in range(16)))
    bvec = b_v[...]
    for cc in range(16):
        z = accs[cc] + bvec
        out_v[pl.ds(cc * 16, 16)] = 1.0 / (1.0 + jnp.exp(-z))

    pltpu.sync_copy(out_v, out_hbm.at[pl.ds(wid * PAIRS_PER_W, PAIRS_PER_W)])


@functools.cache
def _sc_pool():
    mesh = plsc.VectorSubcoreMesh(core_axis_name="c", subcore_axis_name="s")
    return pl.kernel(
        _sc_body,
        mesh=mesh,
        out_type=jax.ShapeDtypeStruct((PAIRS,), jnp.float32),
        scratch_types=[
            pltpu.VMEM((TOK_PER_W,), jnp.int32),
            pltpu.VMEM((TOK_PER_W,), jnp.float32),
            pltpu.VMEM((PAIRS_PER_W,), jnp.float32),
            pltpu.VMEM((16,), jnp.float32),
            pltpu.SemaphoreType.DMA,
        ],
    )


def kernel(X, emb, W, b):
    wr = (W.reshape(2, D).T * (1.0 / L)).astype(jnp.float32)
    t2 = _tc_project(emb, wr).reshape(2 * V)
    q = jnp.arange(TOK_PER_W, dtype=jnp.int32)
    par = (q // (L * PAIRS_PER_W)) * V
    xi = (X.astype(jnp.int32)
          .reshape(NW, PAIRS_PER_W, TOK_PER_PAIR)
          .transpose(0, 2, 1)
          .reshape(NW, TOK_PER_W)) + par[None, :]
    b16 = jnp.broadcast_to(b.astype(jnp.float32), (16,))
    return _sc_pool()(xi, t2, b16)


# speedup vs baseline: 1.0669x; 1.0669x over previous
import functools

import jax
import jax.numpy as jnp
from jax import lax
from jax.experimental import pallas as pl
from jax.experimental.pallas import tpu as pltpu
from jax.experimental.pallas import tpu_sc as plsc

B, L, V, D = 16384, 50, 1000000, 64
NW = 32
PAIRS = B // 2
PAIRS_PER_W = PAIRS // NW
TOK_PER_PAIR = 2 * L
TOK_PER_W = PAIRS_PER_W * TOK_PER_PAIR
CHUNK = 128
N_CHUNKS = TOK_PER_W // CHUNK

_BV = 32768


def _tc_body(w_ref, e_ref, o_ref):
    o_ref[...] = lax.dot_general(
        w_ref[...], e_ref[...], (((0,), (0,)), ((), ())),
        preferred_element_type=jnp.float32)


def _tc_project(emb, wr):
    embT = emb.T
    grid = (V + _BV - 1) // _BV
    return pl.pallas_call(
        _tc_body,
        grid=(grid,),
        in_specs=[
            pl.BlockSpec((D, 2), lambda i: (0, 0)),
            pl.BlockSpec((D, _BV), lambda i: (0, i)),
        ],
        out_specs=pl.BlockSpec((2, _BV), lambda i: (0, i)),
        out_shape=jax.ShapeDtypeStruct((2, V), jnp.float32),
    )(wr, embT)


def _sc_body(x_hbm, t2_hbm, b_hbm, out_hbm,
             idx_v, buf_v, out_v, b_v, sem):
    wid = lax.axis_index("s") * 2 + lax.axis_index("c")
    pltpu.sync_copy(x_hbm.at[wid], idx_v)
    pltpu.sync_copy(b_hbm, b_v)

    n_grp = N_CHUNKS // 8
    ahead = 3

    def _issue8(g):
        for k in range(8):
            c = g * 8 + k
            pltpu.async_copy(
                t2_hbm.at[idx_v.at[pl.ds(c * CHUNK, CHUNK)]],
                buf_v.at[pl.ds(c * CHUNK, CHUNK)], sem)

    for g in range(ahead):
        _issue8(g)

    def _pipe(g, _):
        @pl.when(g + ahead < n_grp)
        def _():
            _issue8(g + ahead)
        for k in range(8):
            c = g * 8 + k
            pltpu.make_async_copy(
                t2_hbm.at[idx_v.at[pl.ds(c * CHUNK, CHUNK)]],
                buf_v.at[pl.ds(c * CHUNK, CHUNK)], sem).wait()
        return 0
    lax.fori_loop(0, n_grp, _pipe, 0)

    def _tok(j, accs):
        base = j * PAIRS_PER_W
        return tuple(a + buf_v[pl.ds(base + cc * 16, 16)]
                     for cc, a in enumerate(accs))

    accs = lax.fori_loop(
        0, TOK_PER_PAIR, _tok,
        tuple(jnp.zeros((16,), jnp.float32) for _ in range(16)))
    bvec = b_v[...]
    for cc in range(16):
        z = accs[cc] + bvec
        out_v[pl.ds(cc * 16, 16)] = 1.0 / (1.0 + jnp.exp(-z))

    pltpu.sync_copy(out_v, out_hbm.at[pl.ds(wid * PAIRS_PER_W, PAIRS_PER_W)])


@functools.cache
def _sc_pool():
    mesh = plsc.VectorSubcoreMesh(core_axis_name="c", subcore_axis_name="s")
    return pl.kernel(
        _sc_body,
        mesh=mesh,
        out_type=jax.ShapeDtypeStruct((PAIRS,), jnp.float32),
        scratch_types=[
            pltpu.VMEM((TOK_PER_W,), jnp.int32),
            pltpu.VMEM((TOK_PER_W,), jnp.float32),
            pltpu.VMEM((PAIRS_PER_W,), jnp.float32),
            pltpu.VMEM((16,), jnp.float32),
            pltpu.SemaphoreType.DMA,
        ],
    )


def kernel(X, emb, W, b):
    wr = (W.reshape(2, D).T * (1.0 / L)).astype(jnp.float32)
    t2 = _tc_project(emb, wr).reshape(2 * V)
    q = jnp.arange(TOK_PER_W, dtype=jnp.int32)
    par = (q // (L * PAIRS_PER_W)) * V
    xi = (X.astype(jnp.int32)
          .reshape(NW, PAIRS_PER_W, TOK_PER_PAIR)
          .transpose(0, 2, 1)
          .reshape(NW, TOK_PER_W)) + par[None, :]
    b16 = jnp.broadcast_to(b.astype(jnp.float32), (16,))
    return _sc_pool()(xi, t2, b16)


# speedup vs baseline: 1.0671x; 1.0002x over previous
import functools

import jax
import jax.numpy as jnp
from jax import lax
from jax.experimental import pallas as pl
from jax.experimental.pallas import tpu as pltpu
from jax.experimental.pallas import tpu_sc as plsc

B, L, V, D = 16384, 50, 1000000, 64
NW = 32
PAIRS = B // 2
PAIRS_PER_W = PAIRS // NW
TOK_PER_PAIR = 2 * L
TOK_PER_W = PAIRS_PER_W * TOK_PER_PAIR
CHUNK = 256
N_CHUNKS = TOK_PER_W // CHUNK

_BV = 32768


def _tc_body(w_ref, e_ref, o_ref):
    o_ref[...] = lax.dot_general(
        w_ref[...], e_ref[...], (((0,), (0,)), ((), ())),
        preferred_element_type=jnp.float32)


def _tc_project(emb, wr):
    embT = emb.T
    grid = (V + _BV - 1) // _BV
    return pl.pallas_call(
        _tc_body,
        grid=(grid,),
        in_specs=[
            pl.BlockSpec((D, 2), lambda i: (0, 0)),
            pl.BlockSpec((D, _BV), lambda i: (0, i)),
        ],
        out_specs=pl.BlockSpec((2, _BV), lambda i: (0, i)),
        out_shape=jax.ShapeDtypeStruct((2, V), jnp.float32),
    )(wr, embT)


def _sc_body(x_hbm, t2_hbm, b_hbm, out_hbm,
             idx_v, buf_v, out_v, b_v, sem):
    wid = lax.axis_index("s") * 2 + lax.axis_index("c")
    pltpu.sync_copy(x_hbm.at[wid], idx_v)
    pltpu.sync_copy(b_hbm, b_v)

    n_grp = N_CHUNKS // 4
    ahead = 3

    def _issue8(g):
        for k in range(4):
            c = g * 4 + k
            pltpu.async_copy(
                t2_hbm.at[idx_v.at[pl.ds(c * CHUNK, CHUNK)]],
                buf_v.at[pl.ds(c * CHUNK, CHUNK)], sem)

    for g in range(ahead):
        _issue8(g)

    def _pipe(g, _):
        @pl.when(g + ahead < n_grp)
        def _():
            _issue8(g + ahead)
        for k in range(4):
            c = g * 4 + k
            pltpu.make_async_copy(
                t2_hbm.at[idx_v.at[pl.ds(c * CHUNK, CHUNK)]],
                buf_v.at[pl.ds(c * CHUNK, CHUNK)], sem).wait()
        return 0
    lax.fori_loop(0, n_grp, _pipe, 0)

    def _tok(j, accs):
        base = j * PAIRS_PER_W
        return tuple(a + buf_v[pl.ds(base + cc * 16, 16)]
                     for cc, a in enumerate(accs))

    accs = lax.fori_loop(
        0, TOK_PER_PAIR, _tok,
        tuple(jnp.zeros((16,), jnp.float32) for _ in range(16)))
    bvec = b_v[...]
    for cc in range(16):
        z = accs[cc] + bvec
        out_v[pl.ds(cc * 16, 16)] = 1.0 / (1.0 + jnp.exp(-z))

    pltpu.sync_copy(out_v, out_hbm.at[pl.ds(wid * PAIRS_PER_W, PAIRS_PER_W)])


@functools.cache
def _sc_pool():
    mesh = plsc.VectorSubcoreMesh(core_axis_name="c", subcore_axis_name="s")
    return pl.kernel(
        _sc_body,
        mesh=mesh,
        out_type=jax.ShapeDtypeStruct((PAIRS,), jnp.float32),
        scratch_types=[
            pltpu.VMEM((TOK_PER_W,), jnp.int32),
            pltpu.VMEM((TOK_PER_W,), jnp.float32),
            pltpu.VMEM((PAIRS_PER_W,), jnp.float32),
            pltpu.VMEM((16,), jnp.float32),
            pltpu.SemaphoreType.DMA,
        ],
    )


def kernel(X, emb, W, b):
    wr = (W.reshape(2, D).T * (1.0 / L)).astype(jnp.float32)
    t2 = _tc_project(emb, wr).reshape(2 * V)
    q = jnp.arange(TOK_PER_W, dtype=jnp.int32)
    par = (q // (L * PAIRS_PER_W)) * V
    xi = (X.astype(jnp.int32)
          .reshape(NW, PAIRS_PER_W, TOK_PER_PAIR)
          .transpose(0, 2, 1)
          .reshape(NW, TOK_PER_W)) + par[None, :]
    b16 = jnp.broadcast_to(b.astype(jnp.float32), (16,))
    return _sc_pool()(xi, t2, b16)
